# baseline (speedup 1.0000x reference)
import functools

import jax
import jax.numpy as jnp
from jax import lax
from jax.experimental import pallas as pl
from jax.experimental.pallas import tpu as pltpu
from jax.experimental.pallas import tpu_sc as plsc

B = 16384
EMB = 64
OUT = 512
N_SINGERS = 417
N_GENRES = 18
N_MUSIC = 42800
NC = 2
NS = 16
NW = NC * NS
BPW = B // NW
CHUNK = 128
NCHUNK = BPW // CHUNK


def _sc_gather_body(em_hbm, idm_hbm, om_hbm, idx_v, rows_v, sem):
    wid = lax.axis_index("s") * NC + lax.axis_index("c")
    base = wid * BPW

    pltpu.sync_copy(idm_hbm.at[pl.ds(base, BPW)], idx_v)
    copies = []
    for j in range(NCHUNK):
        dst = rows_v.at[pl.ds(j * CHUNK, CHUNK)]
        idx = idx_v.at[pl.ds(j * CHUNK, CHUNK)]
        copies.append(pltpu.async_copy(em_hbm.at[idx], dst, sem))
    for c in copies:
        c.wait()
    pltpu.sync_copy(rows_v, om_hbm.at[pl.ds(base, BPW)])


@jax.jit
def _sc_gather(em_view, idm_half):
    mesh = plsc.VectorSubcoreMesh(core_axis_name="c", subcore_axis_name="s",
                                  num_cores=NC, num_subcores=NS)
    k = pl.kernel(_sc_gather_body,
                  out_type=jax.ShapeDtypeStruct((B, 2 * EMB), jnp.float32),
                  mesh=mesh,
                  scratch_types=[
                      pltpu.VMEM((BPW,), jnp.int32),
                      pltpu.VMEM((BPW, 2 * EMB), jnp.float32),
                      pltpu.SemaphoreType.DMA,
                  ],
                  compiler_params=pltpu.CompilerParams(
                      use_tc_tiling_on_sc=False))
    return k(em_view, idm_half)


def _mm_body(m_ref, pidx_ref, sidx_ref, gidx_ref, es_ref, eg_ref,
             w1_ref, w2_ref, w3_ref, b_ref, o_ref):
    bb = m_ref.shape[0]
    mw = m_ref[...]
    par = pidx_ref[0, 0, :]
    m = jnp.where(par[:, None] == 1, mw[:, EMB:], mw[:, :EMB])
    sidx = sidx_ref[0, 0, :]
    gidx = gidx_ref[0, 0, :]
    s_oh = (sidx[:, None] ==
            lax.broadcasted_iota(jnp.int32, (bb, N_SINGERS), 1)
            ).astype(jnp.float32)
    g_oh = (gidx[:, None] ==
            lax.broadcasted_iota(jnp.int32, (bb, N_GENRES), 1)
            ).astype(jnp.float32)
    s_emb = jnp.dot(s_oh, es_ref[...], preferred_element_type=jnp.float32)
    g_emb = jnp.dot(g_oh, eg_ref[...], preferred_element_type=jnp.float32)
    acc = jnp.dot(m, w1_ref[...], preferred_element_type=jnp.float32)
    acc += jnp.dot(s_emb, w2_ref[...], preferred_element_type=jnp.float32)
    acc += jnp.dot(g_emb, w3_ref[...], preferred_element_type=jnp.float32)
    o_ref[...] = acc + b_ref[...]


@functools.partial(jax.jit, static_argnames=("bb",))
def _tc_project(memb, pidx, sidx, gidx, E_singer, E_genre, w1, w2, w3, b,
                bb=2048):
    grid = (B // bb,)
    idx_spec = pl.BlockSpec((1, 1, bb), lambda i: (i, 0, 0))
    w_spec = pl.BlockSpec((EMB, OUT), lambda i: (0, 0))
    return pl.pallas_call(
        _mm_body,
        grid=grid,
        in_specs=[
            pl.BlockSpec((bb, 2 * EMB), lambda i: (i, 0)),
            idx_spec, idx_spec, idx_spec,
            pl.BlockSpec((N_SINGERS, EMB), lambda i: (0, 0)),
            pl.BlockSpec((N_GENRES, EMB), lambda i: (0, 0)),
            w_spec, w_spec, w_spec,
            pl.BlockSpec((1, OUT), lambda i: (0, 0)),
        ],
        out_specs=pl.BlockSpec((bb, OUT), lambda i: (i, 0)),
        out_shape=jax.ShapeDtypeStruct((B, OUT), jnp.float32),
    )(memb, pidx, sidx, gidx, E_singer, E_genre, w1, w2, w3, b)


def kernel(lyric, features, singer, genre, id, W_feat, b_feat,
           E_singer, E_genre, E_music, W_out, b_out):
    bb = 2048
    idm = id.astype(jnp.int32)
    em_view = E_music.reshape(N_MUSIC // 2, 2 * EMB)
    pidx = (idm & 1).reshape(B // bb, 1, bb)
    sidx = singer.astype(jnp.int32).reshape(B // bb, 1, bb)
    gidx = genre.astype(jnp.int32).reshape(B // bb, 1, bb)
    memb = _sc_gather(em_view, idm >> 1)
    WT = W_out.T
    return _tc_project(memb, pidx, sidx, gidx, E_singer, E_genre,
                       WT[:EMB], WT[EMB:2 * EMB], WT[2 * EMB:],
                       b_out.reshape(1, OUT), bb=bb)


# speedup vs baseline: 1.0995x; 1.0995x over previous
import functools

import jax
import jax.numpy as jnp
from jax import lax
from jax.experimental import pallas as pl
from jax.experimental.pallas import tpu as pltpu
from jax.experimental.pallas import tpu_sc as plsc

B = 16384
EMB = 64
OUT = 512
N_SINGERS = 417
N_GENRES = 18
NC = 2
NS = 16
NW = NC * NS
BPW = B // NW
CHUNK = 128
NCHUNK = BPW // CHUNK


def _sc_gather_body(em_hbm, idm_hbm, om_hbm, idx_v, rows_v, sem):
    wid = lax.axis_index("s") * NC + lax.axis_index("c")
    base = wid * BPW

    pltpu.sync_copy(idm_hbm.at[pl.ds(base, BPW)], idx_v)
    copies = []
    for j in range(NCHUNK):
        dst = rows_v.at[pl.ds(j * CHUNK, CHUNK)]
        idx = idx_v.at[pl.ds(j * CHUNK, CHUNK)]
        copies.append(pltpu.async_copy(em_hbm.at[idx], dst, sem))
    for c in copies:
        c.wait()
    pltpu.sync_copy(rows_v,
                    om_hbm.at[pl.ds(base, BPW), pl.ds(0, EMB)])


@jax.jit
def _sc_gather(E_music, idm):
    mesh = plsc.VectorSubcoreMesh(core_axis_name="c", subcore_axis_name="s",
                                  num_cores=NC, num_subcores=NS)
    k = pl.kernel(_sc_gather_body,
                  out_type=jax.ShapeDtypeStruct((B, 2 * EMB), jnp.float32),
                  mesh=mesh,
                  scratch_types=[
                      pltpu.VMEM((BPW,), jnp.int32),
                      pltpu.VMEM((BPW, EMB), jnp.float32),
                      pltpu.SemaphoreType.DMA,
                  ],
                  compiler_params=pltpu.CompilerParams(
                      use_tc_tiling_on_sc=False))
    return k(E_music, idm)


def _mm_body(m_ref, sidx_ref, gidx_ref, es_ref, eg_ref,
             w1_ref, w2_ref, w3_ref, b_ref, o_ref):
    bb = m_ref.shape[0]
    m = m_ref[...][:, :EMB]
    sidx = sidx_ref[0, 0, :]
    gidx = gidx_ref[0, 0, :]
    s_oh = (sidx[:, None] ==
            lax.broadcasted_iota(jnp.int32, (bb, N_SINGERS), 1)
            ).astype(jnp.float32)
    g_oh = (gidx[:, None] ==
            lax.broadcasted_iota(jnp.int32, (bb, N_GENRES), 1)
            ).astype(jnp.float32)
    s_emb = jnp.dot(s_oh, es_ref[...], preferred_element_type=jnp.float32)
    g_emb = jnp.dot(g_oh, eg_ref[...], preferred_element_type=jnp.float32)
    acc = jnp.dot(m, w1_ref[...], preferred_element_type=jnp.float32)
    acc += jnp.dot(s_emb, w2_ref[...], preferred_element_type=jnp.float32)
    acc += jnp.dot(g_emb, w3_ref[...], preferred_element_type=jnp.float32)
    o_ref[...] = acc + b_ref[...]


@functools.partial(jax.jit, static_argnames=("bb",))
def _tc_project(memb, sidx, gidx, E_singer, E_genre, w1, w2, w3, b, bb=2048):
    grid = (B // bb,)
    idx_spec = pl.BlockSpec((1, 1, bb), lambda i: (i, 0, 0))
    w_spec = pl.BlockSpec((EMB, OUT), lambda i: (0, 0))
    return pl.pallas_call(
        _mm_body,
        grid=grid,
        in_specs=[
            pl.BlockSpec((bb, 2 * EMB), lambda i: (i, 0)),
            idx_spec, idx_spec,
            pl.BlockSpec((N_SINGERS, EMB), lambda i: (0, 0)),
            pl.BlockSpec((N_GENRES, EMB), lambda i: (0, 0)),
            w_spec, w_spec, w_spec,
            pl.BlockSpec((1, OUT), lambda i: (0, 0)),
        ],
        out_specs=pl.BlockSpec((bb, OUT), lambda i: (i, 0)),
        out_shape=jax.ShapeDtypeStruct((B, OUT), jnp.float32),
    )(memb, sidx, gidx, E_singer, E_genre, w1, w2, w3, b)


def kernel(lyric, features, singer, genre, id, W_feat, b_feat,
           E_singer, E_genre, E_music, W_out, b_out):
    bb = 2048
    idm = id.astype(jnp.int32)
    sidx = singer.astype(jnp.int32).reshape(B // bb, 1, bb)
    gidx = genre.astype(jnp.int32).reshape(B // bb, 1, bb)
    memb = _sc_gather(E_music, idm)
    WT = W_out.T
    return _tc_project(memb, sidx, gidx, E_singer, E_genre,
                       WT[:EMB], WT[EMB:2 * EMB], WT[2 * EMB:],
                       b_out.reshape(1, OUT), bb=bb)
